# baseline (speedup 1.0000x reference)
import functools

import jax
import jax.numpy as jnp
from jax import lax
from jax.experimental import pallas as pl
from jax.experimental.pallas import tpu as pltpu
from jax.experimental.pallas import tpu_sc as plsc

_EX = 50000
_ST = 50000
_K = 128
_NPAD = 50176
_TBLK = 1024
_G = 16
_PAD_DST = 1 << 22

_f32 = jnp.float32
_i32 = jnp.int32


def _p_body(h_ref, w_ref, a1_ref, p_ref, pmax_ref):
    z = jnp.dot(h_ref[...], w_ref[...].T, preferred_element_type=_f32)
    p = jnp.dot(z, a1_ref[...], preferred_element_type=_f32)
    p_ref[...] = p

    @pl.when(pl.program_id(0) == 0)
    def _():
        pmax_ref[...] = jnp.full((1, 1), -jnp.inf, _f32)

    pmax_ref[...] = jnp.maximum(pmax_ref[...], jnp.max(p))


def _t_body(h_ref, w_ref, p_ref, pmax_ref, tab_ref, wv_ref):
    z = jnp.dot(h_ref[...], w_ref[...].T, preferred_element_type=_f32)
    wv = jnp.exp(p_ref[...] - pmax_ref[...])
    tab_ref[...] = z * wv
    wv_ref[...] = wv[:, 0]


def _prep_layer(h, W, a1, blk, reps=1):
    n = h.shape[0]
    nblk = n // blk
    p, pmax = pl.pallas_call(
        _p_body,
        grid=(nblk,),
        in_specs=[
            pl.BlockSpec((blk, _K), lambda i: (i, 0)),
            pl.BlockSpec((_K, _K), lambda i: (0, 0)),
            pl.BlockSpec((_K, 1), lambda i: (0, 0)),
        ],
        out_specs=[
            pl.BlockSpec((blk, 1), lambda i: (i, 0)),
            pl.BlockSpec((1, 1), lambda i: (0, 0)),
        ],
        out_shape=[
            jax.ShapeDtypeStruct((n, 1), _f32),
            jax.ShapeDtypeStruct((1, 1), _f32),
        ],
    )(h, W, a1)
    if reps > 1:
        assert nblk == 1
        in_i = lambda i: (0, 0)
    else:
        in_i = lambda i: (i, 0)
    tab, wv = pl.pallas_call(
        _t_body,
        grid=(nblk * reps,),
        in_specs=[
            pl.BlockSpec((blk, _K), in_i),
            pl.BlockSpec((_K, _K), lambda i: (0, 0)),
            pl.BlockSpec((blk, 1), in_i),
            pl.BlockSpec((1, 1), lambda i: (0, 0)),
        ],
        out_specs=[
            pl.BlockSpec((blk, _K), lambda i: (i, 0)),
            pl.BlockSpec((blk,), lambda i: (i,)),
        ],
        out_shape=[
            jax.ShapeDtypeStruct((n * reps, _K), _f32),
            jax.ShapeDtypeStruct((n * reps,), _f32),
        ],
    )(h, W, p, pmax)
    return tab, wv


def _add_body(a_ref, b_ref, o_ref):
    o_ref[...] = a_ref[...] + b_ref[...]


def _residual_add(emb, agg, blk):
    n = emb.shape[0]
    return pl.pallas_call(
        _add_body,
        grid=(n // blk,),
        in_specs=[
            pl.BlockSpec((blk, _K), lambda i: (i, 0)),
            pl.BlockSpec((blk, _K), lambda i: (i, 0)),
        ],
        out_specs=pl.BlockSpec((blk, _K), lambda i: (i, 0)),
        out_shape=jax.ShapeDtypeStruct((n, _K), _f32),
    )(emb, agg)


def _exer_body(e_ref, b_ref, c_ref, w1a_ref, w1b_ref, w2a_ref, w2b_ref,
               b1_ref, b2_ref, o_ref):
    e = e_ref[...]
    B = b_ref[...]
    C = c_ref[...]
    s1 = (jnp.dot(e, w1a_ref[...], preferred_element_type=_f32)
          + jnp.dot(B, w1b_ref[...], preferred_element_type=_f32)
          + b1_ref[...])
    s2 = (jnp.dot(e, w2a_ref[...], preferred_element_type=_f32)
          + jnp.dot(C, w2b_ref[...], preferred_element_type=_f32)
          + b2_ref[...])
    mx = jnp.maximum(s1, s2)
    e1 = jnp.exp(s1 - mx)
    e2 = jnp.exp(s2 - mx)
    tot = e1 + e2
    o_ref[...] = e + (e1 / tot) * B + (e2 / tot) * C


def _exer_fuse(exer_emb, B, C, We1, be1, We2, be2, blk):
    n = exer_emb.shape[0]
    w1a = We1[:, :_K].T
    w1b = We1[:, _K:].T
    w2a = We2[:, :_K].T
    w2b = We2[:, _K:].T
    b1 = be1.reshape(1, 1)
    b2 = be2.reshape(1, 1)
    full = lambda i: (0, 0)
    rows = lambda i: (i, 0)
    return pl.pallas_call(
        _exer_body,
        grid=(n // blk,),
        in_specs=[
            pl.BlockSpec((blk, _K), rows),
            pl.BlockSpec((blk, _K), rows),
            pl.BlockSpec((blk, _K), rows),
            pl.BlockSpec((_K, 1), full),
            pl.BlockSpec((_K, 1), full),
            pl.BlockSpec((_K, 1), full),
            pl.BlockSpec((_K, 1), full),
            pl.BlockSpec((1, 1), full),
            pl.BlockSpec((1, 1), full),
        ],
        out_specs=pl.BlockSpec((blk, _K), rows),
        out_shape=jax.ShapeDtypeStruct((n, _K), _f32),
    )(exer_emb, B, C, w1a, w1b, w2a, w2b, b1, b2)


def _static_chunks(total, piece):
    out = []
    left = total
    while left > 0:
        s = min(piece, left)
        out.append(s)
        left -= s
    return out


def _segsum_body(src_base, dst_base, cr, chunks_per_sc, r16, reps,
                 tab, wvec, src_r, dst_r, out_hbm,
                 acc, den, src_blk, dst_blk, kept_s, kept_d,
                 sidx, didx, rows_v, wstage, zrow, dstripe,
                 sem_g0, sem_w0, sem_g1, sem_w1):
    c = lax.axis_index("c")
    s = lax.axis_index("s")
    iota = lax.iota(_i32, 16)
    nblocks = r16 // 16

    def _zb(i, carry):
        for t in range(8):
            zrow[i, pl.ds(t * 16, 16)] = jnp.zeros((16,), _f32)
        return carry

    lax.fori_loop(0, 16, _zb, 0)

    az = cr // 16
    dz = max(16, az)
    act_z = cr // dz
    wo = max(16, cr // 16)
    act_w = cr // wo

    for jc in range(chunks_per_sc):
        chunk_lo = (c * chunks_per_sc + jc) * cr

        off = 0
        for sz in _static_chunks(az, 16):
            pltpu.sync_copy(zrow.at[pl.ds(0, sz)],
                            acc.at[pl.ds(s * az + off, sz)])
            off += sz

        @pl.when(s < act_z)
        def _():
            off2 = 0
            for sz in _static_chunks(dz, 128):
                pltpu.sync_copy(zrow.at[0, pl.ds(0, sz)],
                                den.at[pl.ds(s * dz + off2, sz)])
                off2 += sz

        @pl.when(s == 0)
        def _():
            pltpu.sync_copy(zrow.at[pl.ds(0, _G)], acc.at[pl.ds(cr, _G)])
            pltpu.sync_copy(zrow.at[0, pl.ds(0, _G)], den.at[pl.ds(cr, _G)])

        plsc.subcore_barrier()

        def _block(b, carry):
            row0 = s * r16 + b * 16
            pltpu.sync_copy(src_r.at[pl.ds(row0, 16)], src_blk)
            pltpu.sync_copy(dst_r.at[pl.ds(row0, 16)], dst_blk)

            def _scan_row(r, nk):
                for t in range(8):
                    sv = src_blk[r, pl.ds(t * 16, 16)] - src_base
                    dv = dst_blk[r, pl.ds(t * 16, 16)] - dst_base - chunk_lo
                    m = (dv >= 0) & (dv < cr)
                    mi = m.astype(_i32)
                    cs = plsc.cumsum(mi)
                    pos = nk + cs - mi
                    if reps > 1:
                        sv = sv + 128 * (pos & (reps - 1))
                    plsc.store_scatter(kept_s, [pos], sv, mask=m)
                    plsc.store_scatter(kept_d, [pos], dv, mask=m)
                    nk = nk + plsc.all_reduce_population_count(m)[0]
                return nk

            nk = lax.fori_loop(0, 16, _scan_row, jnp.int32(0))

            for t in range(8):
                plsc.store_scatter(kept_s, [nk + t * 16 + iota], t * 16 + iota)
                plsc.store_scatter(kept_d, [nk + t * 16 + iota], cr + iota)

            nsub = (nk + 127) // 128

            def _mkidx(j):
                ph = j & 1
                for t in range(8):
                    sidx[ph, pl.ds(t * 16, 16)] = kept_s[pl.ds(j * 128 + t * 16, 16)]
                    didx[ph, pl.ds(t * 16, 16)] = kept_d[pl.ds(j * 128 + t * 16, 16)]

            def _fire_g(j):
                @pl.when((j & 1) == 0)
                def _():
                    pltpu.async_copy(tab.at[sidx.at[0]],
                                     rows_v.at[pl.ds(0, 128)], sem_g0)
                    pltpu.async_copy(wvec.at[sidx.at[0]], wstage.at[0], sem_w0)

                @pl.when((j & 1) == 1)
                def _():
                    pltpu.async_copy(tab.at[sidx.at[1]],
                                     rows_v.at[pl.ds(128, 128)], sem_g1)
                    pltpu.async_copy(wvec.at[sidx.at[1]], wstage.at[1], sem_w1)

            @pl.when(nsub > 0)
            def _():
                _mkidx(jnp.int32(0))
                _fire_g(jnp.int32(0))

            def _sub(j, carry2):
                @pl.when(j + 1 < nsub)
                def _():
                    _mkidx(j + 1)
                    _fire_g(j + 1)

                @pl.when((j & 1) == 0)
                def _():
                    pltpu.make_async_copy(tab.at[sidx.at[0]],
                                          rows_v.at[pl.ds(0, 128)],
                                          sem_g0).wait()
                    pltpu.make_async_copy(wvec.at[sidx.at[0]], wstage.at[0],
                                          sem_w0).wait()
                    pltpu.sync_copy(rows_v.at[pl.ds(0, 128)],
                                    acc.at[didx.at[0]], add=True)
                    pltpu.sync_copy(wstage.at[0], den.at[didx.at[0]], add=True)

                @pl.when((j & 1) == 1)
                def _():
                    pltpu.make_async_copy(tab.at[sidx.at[1]],
                                          rows_v.at[pl.ds(128, 128)],
                                          sem_g1).wait()
                    pltpu.make_async_copy(wvec.at[sidx.at[1]], wstage.at[1],
                                          sem_w1).wait()
                    pltpu.sync_copy(rows_v.at[pl.ds(128, 128)],
                                    acc.at[didx.at[1]], add=True)
                    pltpu.sync_copy(wstage.at[1], den.at[didx.at[1]], add=True)
                return carry2

            lax.fori_loop(0, nsub, _sub, 0)
            return carry

        lax.fori_loop(0, nblocks, _block, 0)
        plsc.subcore_barrier()

        @pl.when(s < act_w)
        def _():
            off3 = 0
            for sz in _static_chunks(wo, 128):
                base = s * wo + off3
                pltpu.sync_copy(acc.at[pl.ds(base, sz)], rows_v.at[pl.ds(0, sz)])
                pltpu.sync_copy(den.at[pl.ds(base, sz)], dstripe.at[pl.ds(0, sz)])

                def _rcp(q, carry3):
                    dv = dstripe[pl.ds(q * 16, 16)]
                    dstripe[pl.ds(q * 16, 16)] = jnp.where(
                        dv > 0.0, 1.0 / dv, 0.0)
                    return carry3

                lax.fori_loop(0, sz // 16, _rcp, 0)

                def _divrow(rr, carry3):
                    splat = plsc.load_gather(
                        dstripe, [jnp.full((16,), rr, _i32)])
                    for t in range(8):
                        rows_v[rr, pl.ds(t * 16, 16)] = (
                            rows_v[rr, pl.ds(t * 16, 16)] * splat)
                    return carry3

                lax.fori_loop(0, sz, _divrow, 0)
                pltpu.sync_copy(rows_v.at[pl.ds(0, sz)],
                                out_hbm.at[pl.ds(chunk_lo + base, sz)])
                off3 += sz

        plsc.subcore_barrier()


def _segsum_sc(tab, wvec, src_r, dst_r, src_base, dst_base, cr, chunks_per_sc,
               reps=1):
    nd_out = 2 * chunks_per_sc * cr
    r16 = src_r.shape[0] // 16
    mesh = plsc.VectorSubcoreMesh(core_axis_name="c", subcore_axis_name="s",
                                  num_cores=2, num_subcores=16)
    body = functools.partial(_segsum_body, src_base, dst_base, cr,
                             chunks_per_sc, r16, reps)
    return pl.kernel(
        body,
        out_type=jax.ShapeDtypeStruct((nd_out, _K), _f32),
        mesh=mesh,
        compiler_params=pltpu.CompilerParams(needs_layout_passes=False),
        scratch_types=[
            pltpu.VMEM_SHARED((cr + _G, _K), _f32),
            pltpu.VMEM_SHARED((cr + _G,), _f32),
            pltpu.VMEM((16, 128), _i32),
            pltpu.VMEM((16, 128), _i32),
            pltpu.VMEM((2176,), _i32),
            pltpu.VMEM((2176,), _i32),
            pltpu.VMEM((2, 128), _i32),
            pltpu.VMEM((2, 128), _i32),
            pltpu.VMEM((256, 128), _f32),
            pltpu.VMEM((2, 128), _f32),
            pltpu.VMEM((16, 128), _f32),
            pltpu.VMEM((128,), _f32),
        ] + [pltpu.SemaphoreType.DMA] * 4,
    )(tab, wvec, src_r, dst_r)


def _edge_rows(edges, e_pad):
    e = edges.shape[1]
    src = jnp.concatenate(
        [edges[0].astype(_i32), jnp.zeros((e_pad - e,), _i32)])
    dst = jnp.concatenate(
        [edges[1].astype(_i32), jnp.full((e_pad - e,), _PAD_DST, _i32)])
    return src.reshape(-1, 128), dst.reshape(-1, 128)


def _pad_rows(x, n):
    return jnp.concatenate(
        [x, jnp.zeros((n - x.shape[0], x.shape[1]), x.dtype)], axis=0)


def kernel(kn_emb, exer_emb, all_stu_emb,
           k_from_e_edges, e_from_k_edges, s_from_e_edges, e_from_s_edges,
           W_ke, A_ke, W_ek, A_ek, W_se, A_se, W_es, A_es,
           Wk3, bk3, We1, be1, We2, be2):
    exer_pad = _pad_rows(exer_emb, _NPAD)
    stu_pad = _pad_rows(all_stu_emb, _NPAD)

    a1 = lambda A: A[0, :_K].reshape(_K, 1)

    tab_ke, w_ke = _prep_layer(exer_pad, W_ke, a1(A_ke), _TBLK)
    tab_ek, w_ek = _prep_layer(kn_emb, W_ek, a1(A_ek), _K, reps=16)
    tab_se, w_se = _prep_layer(exer_pad, W_se, a1(A_se), _TBLK)
    tab_es, w_es = _prep_layer(stu_pad, W_es, a1(A_es), _TBLK)

    src_ke, dst_ke = _edge_rows(k_from_e_edges, 524288)
    src_ek, dst_ek = _edge_rows(e_from_k_edges, 524288)
    src_se, dst_se = _edge_rows(s_from_e_edges, 622592)
    src_es, dst_es = _edge_rows(e_from_s_edges, 622592)

    out_ke = _segsum_sc(tab_ke, w_ke, src_ke, dst_ke,
                        src_base=0, dst_base=_EX, cr=64, chunks_per_sc=1)
    out_ek = _segsum_sc(tab_ek, w_ek, src_ek, dst_ek,
                        src_base=_EX, dst_base=0, cr=8448, chunks_per_sc=3,
                        reps=16)
    out_se = _segsum_sc(tab_se, w_se, src_se, dst_se,
                        src_base=0, dst_base=_EX, cr=8448, chunks_per_sc=3)
    out_es = _segsum_sc(tab_es, w_es, src_es, dst_es,
                        src_base=_EX, dst_base=0, cr=8448, chunks_per_sc=3)

    kn_out = _residual_add(kn_emb, out_ke, _K)
    stu_out = _residual_add(all_stu_emb, out_se, 1000)
    exer_out = _exer_fuse(exer_emb, out_ek, out_es,
                          We1, be1, We2, be2, 1000)
    return (kn_out, exer_out, stu_out)


# speedup vs baseline: 1.2598x; 1.2598x over previous
import functools

import jax
import jax.numpy as jnp
from jax import lax
from jax.experimental import pallas as pl
from jax.experimental.pallas import tpu as pltpu
from jax.experimental.pallas import tpu_sc as plsc

_EX = 50000
_ST = 50000
_K = 128
_NPAD = 50176
_TBLK = 1024
_G = 16
_PAD_DST = 1 << 22

_f32 = jnp.float32
_i32 = jnp.int32


def _p_body(h_ref, w_ref, a1_ref, p_ref, pmax_ref):
    z = jnp.dot(h_ref[...], w_ref[...].T, preferred_element_type=_f32)
    p = jnp.dot(z, a1_ref[...], preferred_element_type=_f32)
    p_ref[...] = p

    @pl.when(pl.program_id(0) == 0)
    def _():
        pmax_ref[...] = jnp.full((1, 1), -jnp.inf, _f32)

    pmax_ref[...] = jnp.maximum(pmax_ref[...], jnp.max(p))


def _t_body(h_ref, w_ref, p_ref, pmax_ref, tab_ref, wv_ref):
    z = jnp.dot(h_ref[...], w_ref[...].T, preferred_element_type=_f32)
    wv = jnp.exp(p_ref[...] - pmax_ref[...])
    tab_ref[...] = z * wv
    wv_ref[...] = wv[:, 0]


def _prep_layer(h, W, a1, blk, reps=1):
    n = h.shape[0]
    nblk = n // blk
    p, pmax = pl.pallas_call(
        _p_body,
        grid=(nblk,),
        in_specs=[
            pl.BlockSpec((blk, _K), lambda i: (i, 0)),
            pl.BlockSpec((_K, _K), lambda i: (0, 0)),
            pl.BlockSpec((_K, 1), lambda i: (0, 0)),
        ],
        out_specs=[
            pl.BlockSpec((blk, 1), lambda i: (i, 0)),
            pl.BlockSpec((1, 1), lambda i: (0, 0)),
        ],
        out_shape=[
            jax.ShapeDtypeStruct((n, 1), _f32),
            jax.ShapeDtypeStruct((1, 1), _f32),
        ],
    )(h, W, a1)
    if reps > 1:
        assert nblk == 1
        in_i = lambda i: (0, 0)
    else:
        in_i = lambda i: (i, 0)
    tab, wv = pl.pallas_call(
        _t_body,
        grid=(nblk * reps,),
        in_specs=[
            pl.BlockSpec((blk, _K), in_i),
            pl.BlockSpec((_K, _K), lambda i: (0, 0)),
            pl.BlockSpec((blk, 1), in_i),
            pl.BlockSpec((1, 1), lambda i: (0, 0)),
        ],
        out_specs=[
            pl.BlockSpec((blk, _K), lambda i: (i, 0)),
            pl.BlockSpec((blk,), lambda i: (i,)),
        ],
        out_shape=[
            jax.ShapeDtypeStruct((n * reps, _K), _f32),
            jax.ShapeDtypeStruct((n * reps,), _f32),
        ],
    )(h, W, p, pmax)
    return tab, wv


def _add_body(a_ref, b_ref, o_ref):
    o_ref[...] = a_ref[...] + b_ref[...]


def _residual_add(emb, agg, blk):
    n = emb.shape[0]
    return pl.pallas_call(
        _add_body,
        grid=(n // blk,),
        in_specs=[
            pl.BlockSpec((blk, _K), lambda i: (i, 0)),
            pl.BlockSpec((blk, _K), lambda i: (i, 0)),
        ],
        out_specs=pl.BlockSpec((blk, _K), lambda i: (i, 0)),
        out_shape=jax.ShapeDtypeStruct((n, _K), _f32),
    )(emb, agg)


def _exer_body(e_ref, b_ref, c_ref, w1a_ref, w1b_ref, w2a_ref, w2b_ref,
               b1_ref, b2_ref, o_ref):
    e = e_ref[...]
    B = b_ref[...]
    C = c_ref[...]
    s1 = (jnp.dot(e, w1a_ref[...], preferred_element_type=_f32)
          + jnp.dot(B, w1b_ref[...], preferred_element_type=_f32)
          + b1_ref[...])
    s2 = (jnp.dot(e, w2a_ref[...], preferred_element_type=_f32)
          + jnp.dot(C, w2b_ref[...], preferred_element_type=_f32)
          + b2_ref[...])
    mx = jnp.maximum(s1, s2)
    e1 = jnp.exp(s1 - mx)
    e2 = jnp.exp(s2 - mx)
    tot = e1 + e2
    o_ref[...] = e + (e1 / tot) * B + (e2 / tot) * C


def _exer_fuse(exer_emb, B, C, We1, be1, We2, be2, blk):
    n = exer_emb.shape[0]
    w1a = We1[:, :_K].T
    w1b = We1[:, _K:].T
    w2a = We2[:, :_K].T
    w2b = We2[:, _K:].T
    b1 = be1.reshape(1, 1)
    b2 = be2.reshape(1, 1)
    full = lambda i: (0, 0)
    rows = lambda i: (i, 0)
    return pl.pallas_call(
        _exer_body,
        grid=(n // blk,),
        in_specs=[
            pl.BlockSpec((blk, _K), rows),
            pl.BlockSpec((blk, _K), rows),
            pl.BlockSpec((blk, _K), rows),
            pl.BlockSpec((_K, 1), full),
            pl.BlockSpec((_K, 1), full),
            pl.BlockSpec((_K, 1), full),
            pl.BlockSpec((_K, 1), full),
            pl.BlockSpec((1, 1), full),
            pl.BlockSpec((1, 1), full),
        ],
        out_specs=pl.BlockSpec((blk, _K), rows),
        out_shape=jax.ShapeDtypeStruct((n, _K), _f32),
    )(exer_emb, B, C, w1a, w1b, w2a, w2b, b1, b2)


def _static_chunks(total, piece):
    out = []
    left = total
    while left > 0:
        s = min(piece, left)
        out.append(s)
        left -= s
    return out


def _segsum_body(src_base, dst_base, cr, chunks_per_sc, r16, reps,
                 tab, wvec, src_r, dst_r, out_hbm,
                 acc, den, src_blk, dst_blk, kept_s, kept_d,
                 sidx, didx, rows_v, wstage, zrow, dstripe,
                 sem_g0, sem_w0, sem_g1, sem_w1):
    c = lax.axis_index("c")
    s = lax.axis_index("s")
    iota = lax.iota(_i32, 16)
    nblocks = r16 // 16

    def _zb(i, carry):
        for t in range(8):
            zrow[i, pl.ds(t * 16, 16)] = jnp.zeros((16,), _f32)
        return carry

    lax.fori_loop(0, 16, _zb, 0)

    az = cr // 16
    dz = max(16, az)
    act_z = cr // dz
    wo = max(16, cr // 16)
    act_w = cr // wo

    for jc in range(chunks_per_sc):
        chunk_lo = (c * chunks_per_sc + jc) * cr

        off = 0
        for sz in _static_chunks(az, 16):
            pltpu.sync_copy(zrow.at[pl.ds(0, sz)],
                            acc.at[pl.ds(s * az + off, sz)])
            off += sz

        @pl.when(s < act_z)
        def _():
            off2 = 0
            for sz in _static_chunks(dz, 128):
                pltpu.sync_copy(zrow.at[0, pl.ds(0, sz)],
                                den.at[pl.ds(s * dz + off2, sz)])
                off2 += sz

        @pl.when(s == 0)
        def _():
            pltpu.sync_copy(zrow.at[pl.ds(0, _G)], acc.at[pl.ds(cr, _G)])
            pltpu.sync_copy(zrow.at[0, pl.ds(0, _G)], den.at[pl.ds(cr, _G)])

        plsc.subcore_barrier()

        def _block(b, carry):
            row0 = s * r16 + b * 16
            pltpu.sync_copy(src_r.at[pl.ds(row0, 16)], src_blk)
            pltpu.sync_copy(dst_r.at[pl.ds(row0, 16)], dst_blk)

            def _scan_row(r, nk):
                for t in range(8):
                    sv = src_blk[r, pl.ds(t * 16, 16)] - src_base
                    dv = dst_blk[r, pl.ds(t * 16, 16)] - dst_base - chunk_lo
                    m = (dv >= 0) & (dv < cr)
                    mi = m.astype(_i32)
                    cs = plsc.cumsum(mi)
                    pos = nk + cs - mi
                    if reps > 1:
                        sv = sv + 128 * (pos & (reps - 1))
                    plsc.store_scatter(kept_s, [pos], sv, mask=m)
                    plsc.store_scatter(kept_d, [pos], dv, mask=m)
                    nk = nk + plsc.all_reduce_population_count(m)[0]
                return nk

            nk = lax.fori_loop(0, 16, _scan_row, jnp.int32(0))

            for t in range(4):
                plsc.store_scatter(kept_s, [nk + t * 16 + iota], t * 16 + iota)
                plsc.store_scatter(kept_d, [nk + t * 16 + iota], cr + iota)

            nsub = (nk + 63) // 64

            def _mkidx(j):
                ph = j & 1
                for t in range(4):
                    sidx[ph, pl.ds(t * 16, 16)] = kept_s[pl.ds(j * 64 + t * 16, 16)]
                    didx[ph, pl.ds(t * 16, 16)] = kept_d[pl.ds(j * 64 + t * 16, 16)]

            def _fire_g(j):
                @pl.when((j & 1) == 0)
                def _():
                    pltpu.async_copy(tab.at[sidx.at[0]],
                                     rows_v.at[pl.ds(0, 64)], sem_g0)
                    pltpu.async_copy(wvec.at[sidx.at[0]],
                                     wstage.at[0, pl.ds(0, 64)], sem_w0)

                @pl.when((j & 1) == 1)
                def _():
                    pltpu.async_copy(tab.at[sidx.at[1]],
                                     rows_v.at[pl.ds(64, 64)], sem_g1)
                    pltpu.async_copy(wvec.at[sidx.at[1]],
                                     wstage.at[1, pl.ds(0, 64)], sem_w1)

            @pl.when(nsub > 0)
            def _():
                _mkidx(jnp.int32(0))
                _fire_g(jnp.int32(0))

            def _sub(j, carry2):
                @pl.when(j + 1 < nsub)
                def _():
                    _mkidx(j + 1)
                    _fire_g(j + 1)

                @pl.when((j & 1) == 0)
                def _():
                    pltpu.make_async_copy(tab.at[sidx.at[0]],
                                          rows_v.at[pl.ds(0, 64)],
                                          sem_g0).wait()
                    pltpu.make_async_copy(wvec.at[sidx.at[0]],
                                          wstage.at[0, pl.ds(0, 64)],
                                          sem_w0).wait()
                    pltpu.sync_copy(rows_v.at[pl.ds(0, 64)],
                                    acc.at[didx.at[0]], add=True)
                    pltpu.sync_copy(wstage.at[0, pl.ds(0, 64)],
                                    den.at[didx.at[0]], add=True)

                @pl.when((j & 1) == 1)
                def _():
                    pltpu.make_async_copy(tab.at[sidx.at[1]],
                                          rows_v.at[pl.ds(64, 64)],
                                          sem_g1).wait()
                    pltpu.make_async_copy(wvec.at[sidx.at[1]],
                                          wstage.at[1, pl.ds(0, 64)],
                                          sem_w1).wait()
                    pltpu.sync_copy(rows_v.at[pl.ds(64, 64)],
                                    acc.at[didx.at[1]], add=True)
                    pltpu.sync_copy(wstage.at[1, pl.ds(0, 64)],
                                    den.at[didx.at[1]], add=True)
                return carry2

            lax.fori_loop(0, nsub, _sub, 0)
            return carry

        lax.fori_loop(0, nblocks, _block, 0)
        plsc.subcore_barrier()

        @pl.when(s < act_w)
        def _():
            off3 = 0
            for sz in _static_chunks(wo, 128):
                base = s * wo + off3
                pltpu.sync_copy(acc.at[pl.ds(base, sz)], rows_v.at[pl.ds(0, sz)])
                pltpu.sync_copy(den.at[pl.ds(base, sz)], dstripe.at[pl.ds(0, sz)])

                def _rcp(q, carry3):
                    dv = dstripe[pl.ds(q * 16, 16)]
                    dstripe[pl.ds(q * 16, 16)] = jnp.where(
                        dv > 0.0, 1.0 / dv, 0.0)
                    return carry3

                lax.fori_loop(0, sz // 16, _rcp, 0)

                def _divrow(rr, carry3):
                    splat = plsc.load_gather(
                        dstripe, [jnp.full((16,), rr, _i32)])
                    for t in range(8):
                        rows_v[rr, pl.ds(t * 16, 16)] = (
                            rows_v[rr, pl.ds(t * 16, 16)] * splat)
                    return carry3

                lax.fori_loop(0, sz, _divrow, 0)
                pltpu.sync_copy(rows_v.at[pl.ds(0, sz)],
                                out_hbm.at[pl.ds(chunk_lo + base, sz)])
                off3 += sz

        plsc.subcore_barrier()


def _segsum_sc(tab, wvec, src_r, dst_r, src_base, dst_base, cr, chunks_per_sc,
               reps=1):
    nd_out = 2 * chunks_per_sc * cr
    r16 = src_r.shape[0] // 16
    mesh = plsc.VectorSubcoreMesh(core_axis_name="c", subcore_axis_name="s",
                                  num_cores=2, num_subcores=16)
    body = functools.partial(_segsum_body, src_base, dst_base, cr,
                             chunks_per_sc, r16, reps)
    return pl.kernel(
        body,
        out_type=jax.ShapeDtypeStruct((nd_out, _K), _f32),
        mesh=mesh,
        compiler_params=pltpu.CompilerParams(needs_layout_passes=False),
        scratch_types=[
            pltpu.VMEM_SHARED((cr + _G, _K), _f32),
            pltpu.VMEM_SHARED((cr + _G,), _f32),
            pltpu.VMEM((16, 128), _i32),
            pltpu.VMEM((16, 128), _i32),
            pltpu.VMEM((2176,), _i32),
            pltpu.VMEM((2176,), _i32),
            pltpu.VMEM((2, 64), _i32),
            pltpu.VMEM((2, 64), _i32),
            pltpu.VMEM((128, 128), _f32),
            pltpu.VMEM((2, 64), _f32),
            pltpu.VMEM((16, 128), _f32),
            pltpu.VMEM((128,), _f32),
        ] + [pltpu.SemaphoreType.DMA] * 4,
    )(tab, wvec, src_r, dst_r)


def _edge_rows(edges, e_pad):
    e = edges.shape[1]
    src = jnp.concatenate(
        [edges[0].astype(_i32), jnp.zeros((e_pad - e,), _i32)])
    dst = jnp.concatenate(
        [edges[1].astype(_i32), jnp.full((e_pad - e,), _PAD_DST, _i32)])
    return src.reshape(-1, 128), dst.reshape(-1, 128)


def _pad_rows(x, n):
    return jnp.concatenate(
        [x, jnp.zeros((n - x.shape[0], x.shape[1]), x.dtype)], axis=0)


def kernel(kn_emb, exer_emb, all_stu_emb,
           k_from_e_edges, e_from_k_edges, s_from_e_edges, e_from_s_edges,
           W_ke, A_ke, W_ek, A_ek, W_se, A_se, W_es, A_es,
           Wk3, bk3, We1, be1, We2, be2):
    exer_pad = _pad_rows(exer_emb, _NPAD)
    stu_pad = _pad_rows(all_stu_emb, _NPAD)

    a1 = lambda A: A[0, :_K].reshape(_K, 1)

    tab_ke, w_ke = _prep_layer(exer_pad, W_ke, a1(A_ke), _TBLK)
    tab_ek, w_ek = _prep_layer(kn_emb, W_ek, a1(A_ek), _K, reps=16)
    tab_se, w_se = _prep_layer(exer_pad, W_se, a1(A_se), _TBLK)
    tab_es, w_es = _prep_layer(stu_pad, W_es, a1(A_es), _TBLK)

    src_ke, dst_ke = _edge_rows(k_from_e_edges, 524288)
    src_ek, dst_ek = _edge_rows(e_from_k_edges, 524288)
    src_se, dst_se = _edge_rows(s_from_e_edges, 622592)
    src_es, dst_es = _edge_rows(e_from_s_edges, 622592)

    out_ke = _segsum_sc(tab_ke, w_ke, src_ke, dst_ke,
                        src_base=0, dst_base=_EX, cr=64, chunks_per_sc=1)
    out_ek = _segsum_sc(tab_ek, w_ek, src_ek, dst_ek,
                        src_base=_EX, dst_base=0, cr=12544, chunks_per_sc=2,
                        reps=16)
    out_se = _segsum_sc(tab_se, w_se, src_se, dst_se,
                        src_base=0, dst_base=_EX, cr=12544, chunks_per_sc=2)
    out_es = _segsum_sc(tab_es, w_es, src_es, dst_es,
                        src_base=_EX, dst_base=0, cr=12544, chunks_per_sc=2)

    kn_out = _residual_add(kn_emb, out_ke, _K)
    stu_out = _residual_add(all_stu_emb, out_se, 1000)
    exer_out = _exer_fuse(exer_emb, out_ek, out_es,
                          We1, be1, We2, be2, 1000)
    return (kn_out, exer_out, stu_out)


# speedup vs baseline: 1.3327x; 1.0579x over previous
import functools

import jax
import jax.numpy as jnp
from jax import lax
from jax.experimental import pallas as pl
from jax.experimental.pallas import tpu as pltpu
from jax.experimental.pallas import tpu_sc as plsc

_EX = 50000
_ST = 50000
_K = 128
_NPAD = 50176
_TBLK = 1024
_G = 16
_PAD_DST = 1 << 22

_f32 = jnp.float32
_i32 = jnp.int32


def _p_body(h_ref, w_ref, a1_ref, p_ref, pmax_ref):
    z = jnp.dot(h_ref[...], w_ref[...].T, preferred_element_type=_f32)
    p = jnp.dot(z, a1_ref[...], preferred_element_type=_f32)
    p_ref[...] = p

    @pl.when(pl.program_id(0) == 0)
    def _():
        pmax_ref[...] = jnp.full((1, 1), -jnp.inf, _f32)

    pmax_ref[...] = jnp.maximum(pmax_ref[...], jnp.max(p))


def _t_body(h_ref, w_ref, p_ref, pmax_ref, tab_ref, wv_ref):
    z = jnp.dot(h_ref[...], w_ref[...].T, preferred_element_type=_f32)
    wv = jnp.exp(p_ref[...] - pmax_ref[...])
    tab_ref[...] = z * wv
    wv_ref[...] = wv[:, 0]


def _prep_layer(h, W, a1, blk, reps=1):
    n = h.shape[0]
    nblk = n // blk
    p, pmax = pl.pallas_call(
        _p_body,
        grid=(nblk,),
        in_specs=[
            pl.BlockSpec((blk, _K), lambda i: (i, 0)),
            pl.BlockSpec((_K, _K), lambda i: (0, 0)),
            pl.BlockSpec((_K, 1), lambda i: (0, 0)),
        ],
        out_specs=[
            pl.BlockSpec((blk, 1), lambda i: (i, 0)),
            pl.BlockSpec((1, 1), lambda i: (0, 0)),
        ],
        out_shape=[
            jax.ShapeDtypeStruct((n, 1), _f32),
            jax.ShapeDtypeStruct((1, 1), _f32),
        ],
    )(h, W, a1)
    if reps > 1:
        assert nblk == 1
        in_i = lambda i: (0, 0)
    else:
        in_i = lambda i: (i, 0)
    tab, wv = pl.pallas_call(
        _t_body,
        grid=(nblk * reps,),
        in_specs=[
            pl.BlockSpec((blk, _K), in_i),
            pl.BlockSpec((_K, _K), lambda i: (0, 0)),
            pl.BlockSpec((blk, 1), in_i),
            pl.BlockSpec((1, 1), lambda i: (0, 0)),
        ],
        out_specs=[
            pl.BlockSpec((blk, _K), lambda i: (i, 0)),
            pl.BlockSpec((blk,), lambda i: (i,)),
        ],
        out_shape=[
            jax.ShapeDtypeStruct((n * reps, _K), _f32),
            jax.ShapeDtypeStruct((n * reps,), _f32),
        ],
    )(h, W, p, pmax)
    return tab, wv


def _add_body(a_ref, b_ref, o_ref):
    o_ref[...] = a_ref[...] + b_ref[...]


def _residual_add(emb, agg, blk):
    n = emb.shape[0]
    return pl.pallas_call(
        _add_body,
        grid=(n // blk,),
        in_specs=[
            pl.BlockSpec((blk, _K), lambda i: (i, 0)),
            pl.BlockSpec((blk, _K), lambda i: (i, 0)),
        ],
        out_specs=pl.BlockSpec((blk, _K), lambda i: (i, 0)),
        out_shape=jax.ShapeDtypeStruct((n, _K), _f32),
    )(emb, agg)


def _exer_body(e_ref, b_ref, c_ref, w1a_ref, w1b_ref, w2a_ref, w2b_ref,
               b1_ref, b2_ref, o_ref):
    e = e_ref[...]
    B = b_ref[...]
    C = c_ref[...]
    s1 = (jnp.dot(e, w1a_ref[...], preferred_element_type=_f32)
          + jnp.dot(B, w1b_ref[...], preferred_element_type=_f32)
          + b1_ref[...])
    s2 = (jnp.dot(e, w2a_ref[...], preferred_element_type=_f32)
          + jnp.dot(C, w2b_ref[...], preferred_element_type=_f32)
          + b2_ref[...])
    mx = jnp.maximum(s1, s2)
    e1 = jnp.exp(s1 - mx)
    e2 = jnp.exp(s2 - mx)
    tot = e1 + e2
    o_ref[...] = e + (e1 / tot) * B + (e2 / tot) * C


def _exer_fuse(exer_emb, B, C, We1, be1, We2, be2, blk):
    n = exer_emb.shape[0]
    w1a = We1[:, :_K].T
    w1b = We1[:, _K:].T
    w2a = We2[:, :_K].T
    w2b = We2[:, _K:].T
    b1 = be1.reshape(1, 1)
    b2 = be2.reshape(1, 1)
    full = lambda i: (0, 0)
    rows = lambda i: (i, 0)
    return pl.pallas_call(
        _exer_body,
        grid=(n // blk,),
        in_specs=[
            pl.BlockSpec((blk, _K), rows),
            pl.BlockSpec((blk, _K), rows),
            pl.BlockSpec((blk, _K), rows),
            pl.BlockSpec((_K, 1), full),
            pl.BlockSpec((_K, 1), full),
            pl.BlockSpec((_K, 1), full),
            pl.BlockSpec((_K, 1), full),
            pl.BlockSpec((1, 1), full),
            pl.BlockSpec((1, 1), full),
        ],
        out_specs=pl.BlockSpec((blk, _K), rows),
        out_shape=jax.ShapeDtypeStruct((n, _K), _f32),
    )(exer_emb, B, C, w1a, w1b, w2a, w2b, b1, b2)


def _static_chunks(total, piece):
    out = []
    left = total
    while left > 0:
        s = min(piece, left)
        out.append(s)
        left -= s
    return out


def _segsum_body(src_base, dst_base, cr, chunks_per_sc, r16, reps,
                 tab, wvec, src_r, dst_r, out_hbm,
                 acc, den, src_blk, dst_blk, kept_s, kept_d,
                 sidx, didx, rows_v, wstage, dstripe,
                 sem_g0, sem_w0, sem_g1, sem_w1, sem_es, sem_ed):
    c = lax.axis_index("c")
    s = lax.axis_index("s")
    iota = lax.iota(_i32, 16)
    nblocks = r16 // 16

    az = cr // 16
    dz = max(16, az)
    act_z = cr // dz
    wo = max(16, cr // 16)
    act_w = cr // wo

    for jc in range(chunks_per_sc):
        chunk_lo = (c * chunks_per_sc + jc) * cr

        def _zb(i, zcarry):
            for t in range(8):
                rows_v[i, pl.ds(t * 16, 16)] = jnp.zeros((16,), _f32)
            return zcarry

        lax.fori_loop(0, 128, _zb, 0)

        off = 0
        for sz in _static_chunks(az, 128):
            pltpu.sync_copy(rows_v.at[pl.ds(0, sz)],
                            acc.at[pl.ds(s * az + off, sz)])
            off += sz

        @pl.when(s < act_z)
        def _():
            off2 = 0
            for sz in _static_chunks(dz, 128):
                pltpu.sync_copy(rows_v.at[0, pl.ds(0, sz)],
                                den.at[pl.ds(s * dz + off2, sz)])
                off2 += sz

        @pl.when(s == 0)
        def _():
            pltpu.sync_copy(rows_v.at[pl.ds(0, _G)], acc.at[pl.ds(cr, _G)])
            pltpu.sync_copy(rows_v.at[0, pl.ds(0, _G)], den.at[pl.ds(cr, _G)])

        plsc.subcore_barrier()

        row00 = s * r16
        pltpu.async_copy(src_r.at[pl.ds(row00, 16)], src_blk, sem_es)
        pltpu.async_copy(dst_r.at[pl.ds(row00, 16)], dst_blk, sem_ed)

        def _block(b, carry):
            row0 = s * r16 + b * 16
            pltpu.make_async_copy(src_r.at[pl.ds(row0, 16)], src_blk,
                                  sem_es).wait()
            pltpu.make_async_copy(dst_r.at[pl.ds(row0, 16)], dst_blk,
                                  sem_ed).wait()

            def _scan_row(r, nk):
                for t in range(8):
                    sv = src_blk[r, pl.ds(t * 16, 16)] - src_base
                    dv = dst_blk[r, pl.ds(t * 16, 16)] - dst_base - chunk_lo
                    m = (dv >= 0) & (dv < cr)
                    mi = m.astype(_i32)
                    cs = plsc.cumsum(mi)
                    pos = nk + cs - mi
                    if reps > 1:
                        sv = sv + 128 * (pos & (reps - 1))
                    plsc.store_scatter(kept_s, [pos], sv, mask=m)
                    plsc.store_scatter(kept_d, [pos], dv, mask=m)
                    nk = nk + plsc.all_reduce_population_count(m)[0]
                return nk

            nk = lax.fori_loop(0, 16, _scan_row, jnp.int32(0))

            @pl.when(b + 1 < nblocks)
            def _():
                row1 = s * r16 + (b + 1) * 16
                pltpu.async_copy(src_r.at[pl.ds(row1, 16)], src_blk, sem_es)
                pltpu.async_copy(dst_r.at[pl.ds(row1, 16)], dst_blk, sem_ed)

            for t in range(4):
                plsc.store_scatter(kept_s, [nk + t * 16 + iota], t * 16 + iota)
                plsc.store_scatter(kept_d, [nk + t * 16 + iota], cr + iota)

            nsub = (nk + 63) // 64

            def _mkidx(j):
                ph = j & 1
                for t in range(4):
                    sidx[ph, pl.ds(t * 16, 16)] = kept_s[pl.ds(j * 64 + t * 16, 16)]
                    didx[ph, pl.ds(t * 16, 16)] = kept_d[pl.ds(j * 64 + t * 16, 16)]

            def _fire_g(j):
                @pl.when((j & 1) == 0)
                def _():
                    pltpu.async_copy(tab.at[sidx.at[0]],
                                     rows_v.at[pl.ds(0, 64)], sem_g0)
                    pltpu.async_copy(wvec.at[sidx.at[0]],
                                     wstage.at[0, pl.ds(0, 64)], sem_w0)

                @pl.when((j & 1) == 1)
                def _():
                    pltpu.async_copy(tab.at[sidx.at[1]],
                                     rows_v.at[pl.ds(64, 64)], sem_g1)
                    pltpu.async_copy(wvec.at[sidx.at[1]],
                                     wstage.at[1, pl.ds(0, 64)], sem_w1)

            @pl.when(nsub > 0)
            def _():
                _mkidx(jnp.int32(0))
                _fire_g(jnp.int32(0))

            def _sub(j, carry2):
                @pl.when(j + 1 < nsub)
                def _():
                    _mkidx(j + 1)
                    _fire_g(j + 1)

                @pl.when((j & 1) == 0)
                def _():
                    pltpu.make_async_copy(tab.at[sidx.at[0]],
                                          rows_v.at[pl.ds(0, 64)],
                                          sem_g0).wait()
                    pltpu.make_async_copy(wvec.at[sidx.at[0]],
                                          wstage.at[0, pl.ds(0, 64)],
                                          sem_w0).wait()
                    pltpu.sync_copy(rows_v.at[pl.ds(0, 64)],
                                    acc.at[didx.at[0]], add=True)
                    pltpu.sync_copy(wstage.at[0, pl.ds(0, 64)],
                                    den.at[didx.at[0]], add=True)

                @pl.when((j & 1) == 1)
                def _():
                    pltpu.make_async_copy(tab.at[sidx.at[1]],
                                          rows_v.at[pl.ds(64, 64)],
                                          sem_g1).wait()
                    pltpu.make_async_copy(wvec.at[sidx.at[1]],
                                          wstage.at[1, pl.ds(0, 64)],
                                          sem_w1).wait()
                    pltpu.sync_copy(rows_v.at[pl.ds(64, 64)],
                                    acc.at[didx.at[1]], add=True)
                    pltpu.sync_copy(wstage.at[1, pl.ds(0, 64)],
                                    den.at[didx.at[1]], add=True)
                return carry2

            lax.fori_loop(0, nsub, _sub, 0)
            return carry

        lax.fori_loop(0, nblocks, _block, 0)
        plsc.subcore_barrier()

        @pl.when(s < act_w)
        def _():
            off3 = 0
            for sz in _static_chunks(wo, 128):
                base = s * wo + off3
                pltpu.sync_copy(acc.at[pl.ds(base, sz)], rows_v.at[pl.ds(0, sz)])
                pltpu.sync_copy(den.at[pl.ds(base, sz)], dstripe.at[pl.ds(0, sz)])

                def _rcp(q, carry3):
                    dv = dstripe[pl.ds(q * 16, 16)]
                    dstripe[pl.ds(q * 16, 16)] = jnp.where(
                        dv > 0.0, 1.0 / dv, 0.0)
                    return carry3

                lax.fori_loop(0, sz // 16, _rcp, 0)

                def _divrow(rr, carry3):
                    splat = plsc.load_gather(
                        dstripe, [jnp.full((16,), rr, _i32)])
                    for t in range(8):
                        rows_v[rr, pl.ds(t * 16, 16)] = (
                            rows_v[rr, pl.ds(t * 16, 16)] * splat)
                    return carry3

                lax.fori_loop(0, sz, _divrow, 0)
                pltpu.sync_copy(rows_v.at[pl.ds(0, sz)],
                                out_hbm.at[pl.ds(chunk_lo + base, sz)])
                off3 += sz

        plsc.subcore_barrier()


def _segsum_sc(tab, wvec, src_r, dst_r, src_base, dst_base, cr, chunks_per_sc,
               reps=1):
    nd_out = 2 * chunks_per_sc * cr
    r16 = src_r.shape[0] // 16
    mesh = plsc.VectorSubcoreMesh(core_axis_name="c", subcore_axis_name="s",
                                  num_cores=2, num_subcores=16)
    body = functools.partial(_segsum_body, src_base, dst_base, cr,
                             chunks_per_sc, r16, reps)
    return pl.kernel(
        body,
        out_type=jax.ShapeDtypeStruct((nd_out, _K), _f32),
        mesh=mesh,
        compiler_params=pltpu.CompilerParams(needs_layout_passes=False),
        scratch_types=[
            pltpu.VMEM_SHARED((cr + _G, _K), _f32),
            pltpu.VMEM_SHARED((cr + _G,), _f32),
            pltpu.VMEM((16, 128), _i32),
            pltpu.VMEM((16, 128), _i32),
            pltpu.VMEM((2176,), _i32),
            pltpu.VMEM((2176,), _i32),
            pltpu.VMEM((2, 64), _i32),
            pltpu.VMEM((2, 64), _i32),
            pltpu.VMEM((128, 128), _f32),
            pltpu.VMEM((2, 64), _f32),
            pltpu.VMEM((128,), _f32),
        ] + [pltpu.SemaphoreType.DMA] * 6,
    )(tab, wvec, src_r, dst_r)


def _edge_rows(edges, e_pad):
    e = edges.shape[1]
    src = jnp.concatenate(
        [edges[0].astype(_i32), jnp.zeros((e_pad - e,), _i32)])
    dst = jnp.concatenate(
        [edges[1].astype(_i32), jnp.full((e_pad - e,), _PAD_DST, _i32)])
    return src.reshape(-1, 128), dst.reshape(-1, 128)


def _pad_rows(x, n):
    return jnp.concatenate(
        [x, jnp.zeros((n - x.shape[0], x.shape[1]), x.dtype)], axis=0)


def kernel(kn_emb, exer_emb, all_stu_emb,
           k_from_e_edges, e_from_k_edges, s_from_e_edges, e_from_s_edges,
           W_ke, A_ke, W_ek, A_ek, W_se, A_se, W_es, A_es,
           Wk3, bk3, We1, be1, We2, be2):
    exer_pad = _pad_rows(exer_emb, _NPAD)
    stu_pad = _pad_rows(all_stu_emb, _NPAD)

    a1 = lambda A: A[0, :_K].reshape(_K, 1)

    tab_ke, w_ke = _prep_layer(exer_pad, W_ke, a1(A_ke), _TBLK)
    tab_ek, w_ek = _prep_layer(kn_emb, W_ek, a1(A_ek), _K, reps=16)
    tab_se, w_se = _prep_layer(exer_pad, W_se, a1(A_se), _TBLK)
    tab_es, w_es = _prep_layer(stu_pad, W_es, a1(A_es), _TBLK)

    src_ke, dst_ke = _edge_rows(k_from_e_edges, 524288)
    src_ek, dst_ek = _edge_rows(e_from_k_edges, 524288)
    src_se, dst_se = _edge_rows(s_from_e_edges, 622592)
    src_es, dst_es = _edge_rows(e_from_s_edges, 622592)

    out_ke = _segsum_sc(tab_ke, w_ke, src_ke, dst_ke,
                        src_base=0, dst_base=_EX, cr=64, chunks_per_sc=1)
    out_ek = _segsum_sc(tab_ek, w_ek, src_ek, dst_ek,
                        src_base=_EX, dst_base=0, cr=12544, chunks_per_sc=2,
                        reps=16)
    out_se = _segsum_sc(tab_se, w_se, src_se, dst_se,
                        src_base=0, dst_base=_EX, cr=12544, chunks_per_sc=2)
    out_es = _segsum_sc(tab_es, w_es, src_es, dst_es,
                        src_base=_EX, dst_base=0, cr=12544, chunks_per_sc=2)

    kn_out = _residual_add(kn_emb, out_ke, _K)
    stu_out = _residual_add(all_stu_emb, out_se, 1000)
    exer_out = _exer_fuse(exer_emb, out_ek, out_es,
                          We1, be1, We2, be2, 1000)
    return (kn_out, exer_out, stu_out)
